# baseline (speedup 1.0000x reference)
import functools

import jax
import jax.numpy as jnp
from jax import lax
from jax.experimental import pallas as pl
from jax.experimental.pallas import tpu as pltpu
from jax.experimental.pallas import tpu_sc as plsc

N = 100000
D = 128
S = 1024
L = 16
NC = 2
NS = 16
NW = NC * NS
CHUNK = 256
CD = CHUNK * D
NFULL = N // CHUNK
TAIL = N - NFULL * CHUNK
ACC = S * L

_mesh = plsc.VectorSubcoreMesh(core_axis_name="c", subcore_axis_name="s")
_params = pltpu.CompilerParams(needs_layout_passes=False)


def _wid():
    return lax.axis_index("s") * NC + lax.axis_index("c")


@functools.partial(
    pl.kernel,
    mesh=_mesh,
    out_type=[
        jax.ShapeDtypeStruct((NW * ACC,), jnp.float32),
        jax.ShapeDtypeStruct((NW * ACC,), jnp.float32),
    ],
    scratch_types=[
        pltpu.VMEM((2 * CD,), jnp.float32),
        pltpu.VMEM((2 * CHUNK,), jnp.int32),
        pltpu.VMEM((D,), jnp.float32),
        pltpu.VMEM((ACC,), jnp.float32),
        pltpu.VMEM((ACC,), jnp.float32),
        pltpu.SemaphoreType.DMA,
        pltpu.SemaphoreType.DMA,
    ],
    compiler_params=_params,
)
def _phase1(x_hbm, b_hbm, w_hbm, a_hbm, c_hbm,
            xbuf, bbuf, wbuf, acc, cnt, semx, semb):
    wid = _wid()
    iota = lax.iota(jnp.int32, L)
    zero16 = jnp.zeros((L,), jnp.float32)
    ones16 = jnp.ones((L,), jnp.float32)

    pltpu.sync_copy(w_hbm, wbuf)
    wv = [wbuf[pl.ds(16 * c, 16)] for c in range(8)]

    def zbody(i, _):
        acc[pl.ds(i * 16, 16)] = zero16
        cnt[pl.ds(i * 16, 16)] = zero16
        return 0
    lax.fori_loop(0, S, zbody, 0)

    trips = jnp.where(wid < NFULL % NW, NFULL // NW + 1, NFULL // NW)

    def chunk_rowbase(k):
        return (wid + k * NW) * CHUNK

    def start_dma(k, par):
        rb = chunk_rowbase(k)
        pltpu.async_copy(x_hbm.at[pl.ds(rb * D, CD)],
                         xbuf.at[pl.ds(par * CD, CD)], semx)
        pltpu.async_copy(b_hbm.at[pl.ds(rb, CHUNK)],
                         bbuf.at[pl.ds(par * CHUNK, CHUNK)], semb)

    def process(xoff0, boff0, ngroups, unroll=2):
        iota_s = iota * S
        def one_group(r0):
            bv = bbuf[pl.ds(boff0 + r0, 16)]
            plsc.addupdate_scatter(cnt, [bv + iota_s], ones16)
            for i in range(L):
                bs = jnp.take_along_axis(
                    bv, jnp.full((L,), i, jnp.int32), axis=0,
                    mode="promise_in_bounds")
                xoff = xoff0 + (r0 + i) * D
                p = [xbuf[pl.ds(xoff + c * 16, 16)] * wv[c] for c in range(8)]
                y = ((p[0] + p[1]) + (p[2] + p[3])) + ((p[4] + p[5]) + (p[6] + p[7]))
                plsc.addupdate_scatter(acc, [bs + iota_s], y)

        def gbody(g, _):
            for u in range(unroll):
                one_group((g * unroll + u) * L)
            return 0
        assert ngroups % unroll == 0
        lax.fori_loop(0, ngroups // unroll, gbody, 0)

    start_dma(0, 0)

    def cbody(k, _):
        par = lax.rem(k, 2)
        pltpu.make_async_copy(x_hbm.at[pl.ds(0, CD)],
                              xbuf.at[pl.ds(par * CD, CD)], semx).wait()
        pltpu.make_async_copy(b_hbm.at[pl.ds(0, CHUNK)],
                              bbuf.at[pl.ds(par * CHUNK, CHUNK)], semb).wait()

        @pl.when(k + 1 < trips)
        def _():
            start_dma(k + 1, 1 - par)

        process(par * CD, par * CHUNK, CHUNK // L)
        return 0
    lax.fori_loop(0, trips, cbody, 0)

    @pl.when(wid == NW - 1)
    def _():
        pltpu.sync_copy(x_hbm.at[pl.ds(NFULL * CD, TAIL * D)],
                        xbuf.at[pl.ds(0, TAIL * D)])
        pltpu.sync_copy(b_hbm.at[pl.ds(NFULL * CHUNK, TAIL)],
                        bbuf.at[pl.ds(0, TAIL)])
        process(0, 0, TAIL // L)

    pltpu.sync_copy(acc, a_hbm.at[pl.ds(wid * ACC, ACC)])
    pltpu.sync_copy(cnt, c_hbm.at[pl.ds(wid * ACC, ACC)])


NP = NW * L


def _reduce_body(a_ref, c_ref, b_ref, o_ref):
    sums = jnp.sum(a_ref[...], axis=0)
    cnts = jnp.sum(c_ref[...], axis=0)
    o_ref[...] = sums / jnp.maximum(cnts, 1.0) + b_ref[...]


_reduce = pl.pallas_call(
    _reduce_body,
    in_specs=[
        pl.BlockSpec((NP, S), lambda: (0, 0)),
        pl.BlockSpec((NP, S), lambda: (0, 0)),
        pl.BlockSpec((S,), lambda: (0,)),
    ],
    out_specs=pl.BlockSpec((S,), lambda: (0,)),
    out_shape=jax.ShapeDtypeStruct((S,), jnp.float32),
)


def kernel(x, batch, W, b):
    x1 = x.reshape(-1)
    bi = batch.astype(jnp.int32)
    wv = W.reshape(D).astype(jnp.float32)
    bvec = jnp.broadcast_to(b.astype(jnp.float32), (S,))
    a, c = _phase1(x1, bi, wv)
    return _reduce(a.reshape(NP, S), c.reshape(NP, S), bvec)


# speedup vs baseline: 1.3313x; 1.3313x over previous
import functools

import jax
import jax.numpy as jnp
from jax import lax
from jax.experimental import pallas as pl
from jax.experimental.pallas import tpu as pltpu
from jax.experimental.pallas import tpu_sc as plsc

N = 100000
D = 128
S = 1024
L = 16
NC = 2
NS = 16
NW = NC * NS
CHUNK = 256
CD = CHUNK * D
NFULL = N // CHUNK
TAIL = N - NFULL * CHUNK
ACC = S * L

_mesh = plsc.VectorSubcoreMesh(core_axis_name="c", subcore_axis_name="s")
_params = pltpu.CompilerParams(needs_layout_passes=False)


def _wid():
    return lax.axis_index("s") * NC + lax.axis_index("c")


@functools.partial(
    pl.kernel,
    mesh=_mesh,
    out_type=[
        jax.ShapeDtypeStruct((NW * ACC,), jnp.float32),
        jax.ShapeDtypeStruct((NW * ACC,), jnp.float32),
    ],
    scratch_types=[
        pltpu.VMEM((2 * CD,), jnp.float32),
        pltpu.VMEM((2 * CHUNK,), jnp.int32),
        pltpu.VMEM((D,), jnp.float32),
        pltpu.VMEM((ACC,), jnp.float32),
        pltpu.VMEM((ACC,), jnp.float32),
        pltpu.SemaphoreType.DMA,
        pltpu.SemaphoreType.DMA,
    ],
    compiler_params=_params,
)
def _phase1(x_hbm, b_hbm, w_hbm, a_hbm, c_hbm,
            xbuf, bbuf, wbuf, acc, cnt, semx, semb):
    wid = _wid()
    iota = lax.iota(jnp.int32, L)
    zero16 = jnp.zeros((L,), jnp.float32)
    ones16 = jnp.ones((L,), jnp.float32)

    pltpu.sync_copy(w_hbm, wbuf)
    wv = [wbuf[pl.ds(16 * c, 16)] for c in range(8)]

    def zbody(i, _):
        acc[pl.ds(i * 16, 16)] = zero16
        cnt[pl.ds(i * 16, 16)] = zero16
        return 0
    lax.fori_loop(0, S, zbody, 0)

    trips = jnp.where(wid < NFULL % NW, NFULL // NW + 1, NFULL // NW)

    def chunk_rowbase(k):
        return (wid + k * NW) * CHUNK

    def start_dma(k, par):
        rb = chunk_rowbase(k)
        pltpu.async_copy(x_hbm.at[pl.ds(rb * D, CD)],
                         xbuf.at[pl.ds(par * CD, CD)], semx)
        pltpu.async_copy(b_hbm.at[pl.ds(rb, CHUNK)],
                         bbuf.at[pl.ds(par * CHUNK, CHUNK)], semb)

    def process(xoff0, boff0, ngroups, unroll=2):
        def one_group(r0):
            bv = bbuf[pl.ds(boff0 + r0, 16)]
            idxb = bv * 16
            plsc.addupdate_scatter(cnt, [idxb + iota], ones16)
            for i in range(L):
                bs = jnp.take_along_axis(
                    idxb, jnp.full((L,), i, jnp.int32), axis=0,
                    mode="promise_in_bounds")
                xoff = xoff0 + (r0 + i) * D
                p = [xbuf[pl.ds(xoff + c * 16, 16)] * wv[c] for c in range(8)]
                y = ((p[0] + p[1]) + (p[2] + p[3])) + ((p[4] + p[5]) + (p[6] + p[7]))
                plsc.addupdate_scatter(acc, [bs + iota], y)

        def gbody(g, _):
            for u in range(unroll):
                one_group((g * unroll + u) * L)
            return 0
        assert ngroups % unroll == 0
        lax.fori_loop(0, ngroups // unroll, gbody, 0)

    start_dma(0, 0)

    def cbody(k, _):
        par = lax.rem(k, 2)
        pltpu.make_async_copy(x_hbm.at[pl.ds(0, CD)],
                              xbuf.at[pl.ds(par * CD, CD)], semx).wait()
        pltpu.make_async_copy(b_hbm.at[pl.ds(0, CHUNK)],
                              bbuf.at[pl.ds(par * CHUNK, CHUNK)], semb).wait()

        @pl.when(k + 1 < trips)
        def _():
            start_dma(k + 1, 1 - par)

        process(par * CD, par * CHUNK, CHUNK // L)
        return 0
    lax.fori_loop(0, trips, cbody, 0)

    @pl.when(wid == NW - 1)
    def _():
        pltpu.sync_copy(x_hbm.at[pl.ds(NFULL * CD, TAIL * D)],
                        xbuf.at[pl.ds(0, TAIL * D)])
        pltpu.sync_copy(b_hbm.at[pl.ds(NFULL * CHUNK, TAIL)],
                        bbuf.at[pl.ds(0, TAIL)])
        process(0, 0, TAIL // L)

    pltpu.sync_copy(acc, a_hbm.at[pl.ds(wid * ACC, ACC)])
    pltpu.sync_copy(cnt, c_hbm.at[pl.ds(wid * ACC, ACC)])


NP = NW * L


def _reduce_body(a_ref, c_ref, b_ref, o_ref):
    fold = (jnp.arange(128)[:, None] // 16
            == jnp.arange(8)[None, :]).astype(jnp.float32)
    sums = jnp.sum(a_ref[...], axis=0).reshape(128, 128)
    cnts = jnp.sum(c_ref[...], axis=0).reshape(128, 128)
    s8 = jax.lax.dot_general(sums, fold, (((1,), (0,)), ((), ())),
                             preferred_element_type=jnp.float32)
    c8 = jax.lax.dot_general(cnts, fold, (((1,), (0,)), ((), ())),
                             preferred_element_type=jnp.float32)
    o_ref[...] = s8 / jnp.maximum(c8, 1.0) + b_ref[...]


_reduce = pl.pallas_call(
    _reduce_body,
    in_specs=[
        pl.BlockSpec((NW, ACC), lambda: (0, 0)),
        pl.BlockSpec((NW, ACC), lambda: (0, 0)),
        pl.BlockSpec((128, 8), lambda: (0, 0)),
    ],
    out_specs=pl.BlockSpec((128, 8), lambda: (0, 0)),
    out_shape=jax.ShapeDtypeStruct((128, 8), jnp.float32),
)


def kernel(x, batch, W, b):
    x1 = x.reshape(-1)
    bi = batch.astype(jnp.int32)
    wv = W.reshape(D).astype(jnp.float32)
    b2 = jnp.broadcast_to(b.astype(jnp.float32), (128, 8))
    a, c = _phase1(x1, bi, wv)
    return _reduce(a.reshape(NW, ACC), c.reshape(NW, ACC), b2).reshape(S)
